# baseline (speedup 1.0000x reference)
import functools

import jax
import jax.numpy as jnp
from jax.experimental import pallas as pl
from jax.experimental.pallas import tpu as pltpu


def _gem_body(x_ref, o_ref, *, eps, inv_cols, inv_p):
    x = x_ref[...]
    v = jnp.maximum(x, eps)
    c = v * v * v
    s = jnp.sum(c, axis=1)
    s2 = jnp.sum(s, axis=1, keepdims=True)
    m = s2 * inv_cols
    o_ref[...] = jnp.exp(jnp.log(m) * inv_p)


@functools.partial(jax.jit, static_argnames=("p", "eps"))
def _gem_pool(x, p=3.0, eps=1e-6):
    N, C, H, W = x.shape
    rows = N * C

    x3 = x.reshape(rows, H, W)

    tile_R = min(rows, 1024)
    grid = (pl.cdiv(rows, tile_R),)

    body = functools.partial(
        _gem_body, eps=float(eps), inv_cols=1.0 / float(H * W),
        inv_p=1.0 / float(p))

    y = pl.pallas_call(
        body,
        out_shape=jax.ShapeDtypeStruct((rows, 1), jnp.float32),
        grid=grid,
        in_specs=[pl.BlockSpec((tile_R, H, W), lambda i: (i, 0, 0))],
        out_specs=pl.BlockSpec((tile_R, 1), lambda i: (i, 0)),
        compiler_params=pltpu.CompilerParams(
            dimension_semantics=("parallel",)),
    )(x3)

    return y.reshape(rows).astype(x.dtype).reshape(N, C, 1, 1)


def kernel(x):
    return _gem_pool(x, 3.0, eps=1e-6)


# speedup vs baseline: 6.2785x; 6.2785x over previous
import functools

import jax
import jax.numpy as jnp
from jax.experimental import pallas as pl
from jax.experimental.pallas import tpu as pltpu


def _gem_body(x_ref, o_ref, acc_ref, *, eps, inv_cols, inv_p, n_k):
    k = pl.program_id(1)

    @pl.when(k == 0)
    def _():
        acc_ref[...] = jnp.zeros_like(acc_ref)

    v = jnp.maximum(x_ref[0], eps)
    acc_ref[...] += v * v * v

    @pl.when(k == n_k - 1)
    def _():
        m = acc_ref[...] * inv_cols
        o_ref[...] = jnp.exp(jnp.log(m) * inv_p)


@functools.partial(jax.jit, static_argnames=("p", "eps"))
def _gem_pool(x, p=3.0, eps=1e-6):
    N, C, H, W = x.shape
    HW = H * W

    xt = x.transpose(2, 3, 0, 1).reshape(HW, N, C)

    TN = max(8, N // 2)
    grid = (pl.cdiv(N, TN), HW)

    body = functools.partial(
        _gem_body, eps=float(eps), inv_cols=1.0 / float(HW),
        inv_p=1.0 / float(p), n_k=HW)

    y = pl.pallas_call(
        body,
        out_shape=jax.ShapeDtypeStruct((N, C), jnp.float32),
        grid=grid,
        in_specs=[pl.BlockSpec((1, TN, C), lambda i, k: (k, i, 0))],
        out_specs=pl.BlockSpec((TN, C), lambda i, k: (i, 0)),
        scratch_shapes=[pltpu.VMEM((TN, C), jnp.float32)],
        compiler_params=pltpu.CompilerParams(
            dimension_semantics=("parallel", "arbitrary")),
    )(xt)

    return y.astype(x.dtype).reshape(N, C, 1, 1)


def kernel(x):
    return _gem_pool(x, 3.0, eps=1e-6)


# speedup vs baseline: 20.0792x; 3.1981x over previous
import functools

import jax
import jax.numpy as jnp
from jax.experimental import pallas as pl
from jax.experimental.pallas import tpu as pltpu


def _gem_body(x_ref, o_ref, acc_ref, *, eps, inv_cols, inv_p, n_k):
    k = pl.program_id(1)

    @pl.when(k == 0)
    def _():
        acc_ref[...] = jnp.zeros_like(acc_ref)

    v = jnp.maximum(x_ref[...], eps)
    acc_ref[...] += jnp.sum(v * v * v, axis=0)

    @pl.when(k == n_k - 1)
    def _():
        m = acc_ref[...] * inv_cols
        o_ref[...] = jnp.exp(jnp.log(m) * inv_p)


@functools.partial(jax.jit, static_argnames=("p", "eps"))
def _gem_pool(x, p=3.0, eps=1e-6):
    N, C, H, W = x.shape
    HW = H * W

    xt = x.transpose(2, 3, 0, 1).reshape(HW, N, C)

    TN = max(8, N // 2)
    KB = 7
    grid = (pl.cdiv(N, TN), pl.cdiv(HW, KB))

    body = functools.partial(
        _gem_body, eps=float(eps), inv_cols=1.0 / float(HW),
        inv_p=1.0 / float(p), n_k=grid[1])

    y = pl.pallas_call(
        body,
        out_shape=jax.ShapeDtypeStruct((N, C), jnp.float32),
        grid=grid,
        in_specs=[pl.BlockSpec((KB, TN, C), lambda i, k: (k, i, 0))],
        out_specs=pl.BlockSpec((TN, C), lambda i, k: (i, 0)),
        scratch_shapes=[pltpu.VMEM((TN, C), jnp.float32)],
        compiler_params=pltpu.CompilerParams(
            dimension_semantics=("parallel", "arbitrary")),
    )(xt)

    return y.astype(x.dtype).reshape(N, C, 1, 1)


def kernel(x):
    return _gem_pool(x, 3.0, eps=1e-6)
